# baseline (speedup 1.0000x reference)
import functools

import jax
import jax.numpy as jnp
from jax import lax
from jax.experimental import pallas as pl

NSEL = 300
ROWLEN = 900 * 91
RPAD = 81920
NR = 640
NLANE = 128
HI = 19
LO = 16
OPAD = 384
I32MIN = -2147483648


def _prefix_exclusive(m, tri_lane, tri_blk):
    mf = m.astype(jnp.float32)
    p1 = lax.dot_general(mf, tri_lane, (((1,), (0,)), ((), ())),
                         precision=lax.Precision.HIGHEST)
    bs = p1[:, NLANE - 1:NLANE]
    boff = lax.dot_general(tri_blk, bs, (((1,), (0,)), ((), ())),
                           precision=lax.Precision.HIGHEST)
    return p1 - mf + boff


def _postprocess_kernel(keys_ref, boxes_ref, scale_ref,
                        scores_ref, labels_ref, oboxes_ref):
    k = keys_ref[0]
    n300 = jnp.int32(NSEL)

    nonneg = k >= 0
    c0 = jnp.sum(nonneg.astype(jnp.int32))
    not_s = c0 >= n300
    base = jnp.where(not_s, jnp.int32(0), c0)
    elig = nonneg == not_s
    ue = jnp.where(elig, k & jnp.int32(0x7FFFFFFF), jnp.int32(-1))

    def bit_round(i, t):
        cand = t | lax.shift_left(jnp.int32(1), jnp.int32(30) - i)
        cnt = base + jnp.sum((ue >= cand).astype(jnp.int32))
        return jnp.where(cnt >= n300, cand, t)
    t = lax.fori_loop(0, 31, bit_round, jnp.int32(0))
    vstar = jnp.where(not_s, t, t | jnp.int32(I32MIN))

    m_gt = k > vstar
    g = jnp.sum(m_gt.astype(jnp.int32))
    need = (n300 - g).astype(jnp.float32)
    m_eq = k == vstar

    iota_l = lax.broadcasted_iota(jnp.int32, (NLANE, NLANE), 0)
    iota_l2 = lax.broadcasted_iota(jnp.int32, (NLANE, NLANE), 1)
    tri_lane = (iota_l <= iota_l2).astype(jnp.float32)
    iota_b = lax.broadcasted_iota(jnp.int32, (NR, NR), 0)
    iota_b2 = lax.broadcasted_iota(jnp.int32, (NR, NR), 1)
    tri_blk = (iota_b2 < iota_b).astype(jnp.float32)

    pex_eq = _prefix_exclusive(m_eq, tri_lane, tri_blk)
    m = m_gt | (m_eq & (pex_eq < need))
    pos = _prefix_exclusive(m, tri_lane, tri_blk).astype(jnp.int32)

    hi = lax.shift_right_logical(pos, 4)
    lo = pos & jnp.int32(LO - 1)
    ioh = lax.broadcasted_iota(jnp.int32, (NR, HI, NLANE), 1)
    iol = lax.broadcasted_iota(jnp.int32, (NR, LO, NLANE), 1)
    a_oh = ((hi[:, None, :] == ioh) & m[:, None, :]).astype(jnp.float32)
    b_oh = (lo[:, None, :] == iol).astype(jnp.float32)

    kh = lax.shift_right_arithmetic(k, 16).astype(jnp.float32)
    kl = (k & jnp.int32(0xFFFF)).astype(jnp.float32)
    fidx = (lax.broadcasted_iota(jnp.int32, (NR, NLANE), 0) * NLANE
            + lax.broadcasted_iota(jnp.int32, (NR, NLANE), 1)
            ).astype(jnp.float32)

    def compact(p):
        bw = b_oh * p[:, None, :]
        c = lax.dot_general(a_oh, bw, (((2,), (2,)), ((0,), (0,))),
                            precision=lax.Precision.HIGHEST)
        return jnp.sum(c, axis=0)
    ckh = compact(kh)
    ckl = compact(kl)
    cidx = compact(fidx)

    slot = (lax.broadcasted_iota(jnp.int32, (HI, LO), 0) * LO
            + lax.broadcasted_iota(jnp.int32, (HI, LO), 1))
    valid = slot < NSEL
    skh = jnp.where(valid, ckh, jnp.float32(-65536.0))
    skl = jnp.where(valid, ckl, jnp.float32(0.0))
    sidx = jnp.where(valid, cidx, jnp.float32(1e9))

    ah = skh[:, :, None, None]
    al = skl[:, :, None, None]
    ai = sidx[:, :, None, None]
    bh = skh[None, None, :, :]
    bl = skl[None, None, :, :]
    bi = sidx[None, None, :, :]
    o_gt = (bh > ah) | ((bh == ah) & (bl > al))
    o_eq = (bh == ah) & (bl == al) & (bi < ai)
    rnk = jnp.sum((o_gt | o_eq).astype(jnp.int32), axis=(2, 3))

    ior = lax.broadcasted_iota(jnp.int32, (HI, LO, OPAD), 2)
    perm = (rnk[:, :, None] == ior).astype(jnp.float32)

    def extract(c):
        return jnp.sum(c[:, :, None] * perm, axis=(0, 1))
    okh = extract(skh).astype(jnp.int32)
    okl = extract(skl).astype(jnp.int32)
    oidxf = extract(sidx)

    ks = lax.shift_left(okh, 16) | okl
    sbits = ks ^ (lax.shift_right_arithmetic(ks, 31)
                  & jnp.int32(0x7FFFFFFF))
    logit = lax.bitcast_convert_type(sbits, jnp.float32)
    scores_ref[0, 0:1, :] = jax.nn.sigmoid(logit)[None, :]

    oidx = oidxf.astype(jnp.int32)
    q0 = (oidxf * (1.0 / 91.0)).astype(jnp.int32)
    r0 = oidx - q0 * 91
    q1 = jnp.where(r0 < 0, q0 - 1, q0)
    r1 = jnp.where(r0 < 0, r0 + 91, r0)
    qi = jnp.where(r1 >= 91, q1 + 1, q1)
    labels_ref[0, 0:1, :] = jnp.where(r1 >= 91, r1 - 91, r1)[None, :]

    bx = boxes_ref[0]
    xc = bx[:, 0:1]
    yc = bx[:, 1:2]
    w = bx[:, 2:3]
    h = bx[:, 3:4]
    xyxy = jnp.concatenate(
        [xc - 0.5 * w, yc - 0.5 * h, xc + 0.5 * w, yc + 0.5 * h], axis=1)
    ioq = lax.broadcasted_iota(jnp.int32, (OPAD, bx.shape[0]), 1)
    qoh = (qi[:, None] == ioq).astype(jnp.float32)
    gb = lax.dot_general(qoh, xyxy, (((1,), (0,)), ((), ())),
                         precision=lax.Precision.HIGHEST)
    sc = scale_ref[pl.ds(pl.program_id(0), 1), :]
    oboxes_ref[0] = gb * sc


def kernel(pred_logits, pred_boxes, target_sizes):
    B, Q, C = pred_logits.shape
    bits = lax.bitcast_convert_type(pred_logits, jnp.int32).reshape(B, Q * C)
    keys = bits ^ (lax.shift_right_arithmetic(bits, 31)
                   & jnp.int32(0x7FFFFFFF))
    keys = jnp.pad(keys, ((0, 0), (0, RPAD - ROWLEN)),
                   constant_values=jnp.iinfo(jnp.int32).min)
    keys = keys.reshape(B, NR, NLANE)
    img_h = target_sizes[:, 0].astype(jnp.float32)
    img_w = target_sizes[:, 1].astype(jnp.float32)
    scale_fct = jnp.stack([img_w, img_h, img_w, img_h], axis=1)

    scores, labels, boxes = pl.pallas_call(
        _postprocess_kernel,
        grid=(B,),
        in_specs=[
            pl.BlockSpec((1, NR, NLANE), lambda i: (i, 0, 0)),
            pl.BlockSpec((1, Q, 4), lambda i: (i, 0, 0)),
            pl.BlockSpec((B, 4), lambda i: (0, 0)),
        ],
        out_specs=[
            pl.BlockSpec((1, 8, OPAD), lambda i: (i, 0, 0)),
            pl.BlockSpec((1, 8, OPAD), lambda i: (i, 0, 0)),
            pl.BlockSpec((1, OPAD, 4), lambda i: (i, 0, 0)),
        ],
        out_shape=[
            jax.ShapeDtypeStruct((B, 8, OPAD), jnp.float32),
            jax.ShapeDtypeStruct((B, 8, OPAD), jnp.int32),
            jax.ShapeDtypeStruct((B, OPAD, 4), jnp.float32),
        ],
    )(keys, pred_boxes, scale_fct)
    return scores[:, 0, :NSEL], labels[:, 0, :NSEL], boxes[:, :NSEL, :]
